# Initial kernel scaffold; baseline (speedup 1.0000x reference)
import jax
import jax.numpy as jnp
from jax.experimental import pallas as pl


def kernel(x, edge_index, batch, W1, b1, W2, b2, W3, b3, L1, bl1, L2, bl2):
    raise NotImplementedError("write your pallas kernel here")


# speedup vs baseline: 13.2174x; 13.2174x over previous
import functools

import jax
import jax.numpy as jnp
from jax import lax
from jax.experimental import pallas as pl
from jax.experimental.pallas import tpu as pltpu
from jax.experimental.pallas import tpu_sc as plsc

N = 10000
E = 320000
D = 128
H = 128
G = 128

NP = 10240
EP = 323584
NC = 2
NS = 16
NW = NC * NS
EDGES_PER_TILE = EP // NW
CHUNK = 128
NCHUNK = EDGES_PER_TILE // CHUNK
ROWS_PER_TILE = NP // NS
BN = 1024
NBLK = NP // BN


def _sc_mesh():
    return plsc.VectorSubcoreMesh(core_axis_name="c", subcore_axis_name="s",
                                  num_cores=NC, num_subcores=NS)


@functools.cache
def _get_sc_degree():
    @functools.partial(
        pl.kernel,
        mesh=_sc_mesh(),
        out_type=jax.ShapeDtypeStruct((NC, NP, H), jnp.float32),
        scratch_types=[
            pltpu.VMEM((CHUNK,), jnp.int32),
            pltpu.VMEM((CHUNK, H), jnp.float32),
            pltpu.VMEM_SHARED((NP, H), jnp.float32),
        ],
    )
    def body(dst_hbm, ones_hbm, zeros_hbm, out_hbm, dst_v, ones_v, acc_sh):
        cid = lax.axis_index("c")
        sid = lax.axis_index("s")
        wid = sid * NC + cid
        row0 = sid * ROWS_PER_TILE
        pltpu.sync_copy(zeros_hbm.at[pl.ds(row0, ROWS_PER_TILE)],
                        acc_sh.at[pl.ds(row0, ROWS_PER_TILE)])
        pltpu.sync_copy(ones_hbm, ones_v)
        plsc.subcore_barrier()
        base = wid * EDGES_PER_TILE

        def step(i, carry):
            off = base + i * CHUNK
            pltpu.sync_copy(dst_hbm.at[pl.ds(off, CHUNK)], dst_v)
            pltpu.sync_copy(ones_v, acc_sh.at[dst_v], add=True)
            return carry

        lax.fori_loop(0, NCHUNK, step, 0)
        plsc.subcore_barrier()
        pltpu.sync_copy(acc_sh.at[pl.ds(row0, ROWS_PER_TILE)],
                        out_hbm.at[cid, pl.ds(row0, ROWS_PER_TILE)])

    return body


def _sc_degree(dstp, ones, zeros):
    return _get_sc_degree()(dstp, ones, zeros)


@functools.cache
def _get_sc_scatter():
    @functools.partial(
        pl.kernel,
        mesh=_sc_mesh(),
        out_type=jax.ShapeDtypeStruct((NC, NP, H), jnp.float32),
        scratch_types=[
            pltpu.VMEM((CHUNK,), jnp.int32),
            pltpu.VMEM((CHUNK,), jnp.int32),
            pltpu.VMEM((CHUNK, H), jnp.float32),
            pltpu.VMEM_SHARED((NP, H), jnp.float32),
            pltpu.SemaphoreType.DMA,
        ],
    )
    def body(ht_hbm, src_hbm, dst_hbm, zeros_hbm, out_hbm,
             src_v, dst_v, rows_v, acc_sh, sem):
        cid = lax.axis_index("c")
        sid = lax.axis_index("s")
        wid = sid * NC + cid
        row0 = sid * ROWS_PER_TILE
        pltpu.sync_copy(zeros_hbm.at[pl.ds(row0, ROWS_PER_TILE)],
                        acc_sh.at[pl.ds(row0, ROWS_PER_TILE)])
        plsc.subcore_barrier()
        base = wid * EDGES_PER_TILE

        def step(i, carry):
            off = base + i * CHUNK
            pltpu.sync_copy(src_hbm.at[pl.ds(off, CHUNK)], src_v)
            pltpu.sync_copy(dst_hbm.at[pl.ds(off, CHUNK)], dst_v)
            pltpu.async_copy(ht_hbm.at[src_v], rows_v, sem).wait()
            pltpu.sync_copy(rows_v, acc_sh.at[dst_v], add=True)
            return carry

        lax.fori_loop(0, NCHUNK, step, 0)
        plsc.subcore_barrier()
        pltpu.sync_copy(acc_sh.at[pl.ds(row0, ROWS_PER_TILE)],
                        out_hbm.at[cid, pl.ds(row0, ROWS_PER_TILE)])

    return body


def _sc_scatter(ht, srcp, dstp, zeros):
    return _get_sc_scatter()(ht, srcp, dstp, zeros)


def _tc_first_body(cnt_ref, x_ref, W_ref, dinv_ref, ht_ref):
    cnt = cnt_ref[0, :, 0:1] + cnt_ref[1, :, 0:1]
    dinv = lax.rsqrt(cnt + 1.0)
    dinv_b = jnp.broadcast_to(dinv, (BN, H))
    dinv_ref[...] = dinv_b
    ht_ref[...] = dinv_b * jnp.dot(x_ref[...], W_ref[...],
                                   preferred_element_type=jnp.float32)


def _tc_first(cnt_parts, xp, W1):
    return pl.pallas_call(
        _tc_first_body,
        grid=(NBLK,),
        in_specs=[
            pl.BlockSpec((NC, BN, H), lambda i: (0, i, 0)),
            pl.BlockSpec((BN, H), lambda i: (i, 0)),
            pl.BlockSpec((H, H), lambda i: (0, 0)),
        ],
        out_specs=[
            pl.BlockSpec((BN, H), lambda i: (i, 0)),
            pl.BlockSpec((BN, H), lambda i: (i, 0)),
        ],
        out_shape=[
            jax.ShapeDtypeStruct((NP, H), jnp.float32),
            jax.ShapeDtypeStruct((NP, H), jnp.float32),
        ],
    )(cnt_parts, xp, W1)


def _tc_layer_body(acc_ref, ht_ref, dinv_ref, b_ref, W_ref, out_ref):
    acc = acc_ref[0] + acc_ref[1]
    h = jnp.maximum(dinv_ref[...] * (acc + ht_ref[...]) + b_ref[0:1, :], 0.0)
    out_ref[...] = dinv_ref[...] * jnp.dot(h, W_ref[...],
                                           preferred_element_type=jnp.float32)


def _tc_layer(acc_parts, ht, dinv_b, b2d, Wn):
    return pl.pallas_call(
        _tc_layer_body,
        grid=(NBLK,),
        in_specs=[
            pl.BlockSpec((NC, BN, H), lambda i: (0, i, 0)),
            pl.BlockSpec((BN, H), lambda i: (i, 0)),
            pl.BlockSpec((BN, H), lambda i: (i, 0)),
            pl.BlockSpec((8, H), lambda i: (0, 0)),
            pl.BlockSpec((H, H), lambda i: (0, 0)),
        ],
        out_specs=pl.BlockSpec((BN, H), lambda i: (i, 0)),
        out_shape=jax.ShapeDtypeStruct((NP, H), jnp.float32),
    )(acc_parts, ht, dinv_b, b2d, Wn)


def _tc_head_body(acc_ref, ht_ref, dinv_ref, b_ref, batch_ref,
                  L1_ref, bl1_ref, L2_ref, bl2_ref, out_ref,
                  sums_ref, cnts_ref):
    i = pl.program_id(0)

    @pl.when(i == 0)
    def _():
        sums_ref[...] = jnp.zeros_like(sums_ref)
        cnts_ref[...] = jnp.zeros_like(cnts_ref)

    acc = acc_ref[0] + acc_ref[1]
    h = jnp.maximum(dinv_ref[...] * (acc + ht_ref[...]) + b_ref[0:1, :], 0.0)
    b_ids = batch_ref[0]
    gids = lax.broadcasted_iota(jnp.int32, (G, 1), 0)
    P = (jnp.broadcast_to(b_ids, (G, BN)) == gids).astype(jnp.float32)
    sums_ref[...] += jnp.dot(P, h, preferred_element_type=jnp.float32)
    cnts_ref[...] += jnp.broadcast_to(
        jnp.sum(P, axis=1, keepdims=True), (G, H))

    @pl.when(i == NBLK - 1)
    def _():
        g = sums_ref[...] / jnp.maximum(cnts_ref[...], 1.0)
        z = jnp.maximum(jnp.dot(g, L1_ref[...],
                                preferred_element_type=jnp.float32)
                        + bl1_ref[0:1, :], 0.0)
        out_ref[...] = jnp.dot(z, L2_ref[...],
                               preferred_element_type=jnp.float32) \
            + bl2_ref[0:1, :]


def _tc_head(acc_parts, ht, dinv_b, b2d, batch3, L1, bl1_2d, L2b, bl2_2d):
    return pl.pallas_call(
        _tc_head_body,
        grid=(NBLK,),
        in_specs=[
            pl.BlockSpec((NC, BN, H), lambda i: (0, i, 0)),
            pl.BlockSpec((BN, H), lambda i: (i, 0)),
            pl.BlockSpec((BN, H), lambda i: (i, 0)),
            pl.BlockSpec((8, H), lambda i: (0, 0)),
            pl.BlockSpec((1, 1, BN), lambda i: (i, 0, 0)),
            pl.BlockSpec((H, H), lambda i: (0, 0)),
            pl.BlockSpec((8, H), lambda i: (0, 0)),
            pl.BlockSpec((H, H), lambda i: (0, 0)),
            pl.BlockSpec((8, H), lambda i: (0, 0)),
        ],
        out_specs=pl.BlockSpec((G, H), lambda i: (0, 0)),
        out_shape=jax.ShapeDtypeStruct((G, H), jnp.float32),
        scratch_shapes=[
            pltpu.VMEM((G, H), jnp.float32),
            pltpu.VMEM((G, H), jnp.float32),
        ],
    )(acc_parts, ht, dinv_b, b2d, batch3, L1, bl1_2d, L2b, bl2_2d)


def kernel(x, edge_index, batch, W1, b1, W2, b2, W3, b3, L1, bl1, L2, bl2):
    f32 = jnp.float32
    npad = NP - N
    epad = EP - E
    xp = jnp.concatenate([x.astype(f32), jnp.zeros((npad, D), f32)], axis=0)
    pad_rows = (N + (jnp.arange(epad, dtype=jnp.int32) % npad))
    srcp = jnp.concatenate([edge_index[0], pad_rows])
    dstp = jnp.concatenate([edge_index[1], pad_rows])
    batchp = jnp.concatenate(
        [batch, jnp.full((npad,), G, jnp.int32)]).reshape(NBLK, 1, BN)
    zeros = jnp.zeros((NP, H), f32)
    ones = jnp.ones((CHUNK, H), f32)
    b1_2d = jnp.broadcast_to(b1.reshape(1, H), (8, H))
    b2_2d = jnp.broadcast_to(b2.reshape(1, H), (8, H))
    b3_2d = jnp.broadcast_to(b3.reshape(1, H), (8, H))
    bl1_2d = jnp.broadcast_to(bl1.reshape(1, H), (8, H))
    L2b = jnp.broadcast_to(L2.reshape(H, 1), (H, H))
    bl2_2d = jnp.broadcast_to(bl2.reshape(1, 1), (8, H))

    cnt_parts = _sc_degree(dstp, ones, zeros)

    dinv_b, ht = _tc_first(cnt_parts, xp, W1)
    acc = _sc_scatter(ht, srcp, dstp, zeros)
    ht = _tc_layer(acc, ht, dinv_b, b1_2d, W2)
    acc = _sc_scatter(ht, srcp, dstp, zeros)
    ht = _tc_layer(acc, ht, dinv_b, b2_2d, W3)
    acc = _sc_scatter(ht, srcp, dstp, zeros)
    out = _tc_head(acc, ht, dinv_b, b3_2d, batchp,
                   L1, bl1_2d, L2b, bl2_2d)
    return out[:, 0]
